# Initial kernel scaffold; baseline (speedup 1.0000x reference)
import jax
import jax.numpy as jnp
from jax.experimental import pallas as pl


def kernel(s, vec, edge_indexes, edge_vector, edge_distance, edge_rbf, cutoff_dist, W1, b1, W2, b2, Wr, br):
    raise NotImplementedError("write your pallas kernel here")


# speedup vs baseline: 11.9270x; 11.9270x over previous
import functools

import jax
import jax.numpy as jnp
import numpy as np
from jax import lax
from jax.experimental import pallas as pl
from jax.experimental.pallas import tpu as pltpu
from jax.experimental.pallas import tpu_sc as plsc

N = 10000
E = 160000
F = 128
F3 = 3 * F


_PHI_BLK = 1000


def _phi_body(s_ref, w1_ref, b1_ref, w2_ref, b2_ref, o_ref):
    h = jnp.dot(s_ref[...], w1_ref[...], preferred_element_type=jnp.float32)
    h = jax.nn.silu(h + b1_ref[...])
    o_ref[...] = (
        jnp.dot(h, w2_ref[...], preferred_element_type=jnp.float32) + b2_ref[...]
    )


def _phi_tc(s, W1, b1, W2, b2):
    return pl.pallas_call(
        _phi_body,
        grid=(N // _PHI_BLK,),
        in_specs=[
            pl.BlockSpec((_PHI_BLK, F), lambda i: (i, 0)),
            pl.BlockSpec((F, F), lambda i: (0, 0)),
            pl.BlockSpec((1, F), lambda i: (0, 0)),
            pl.BlockSpec((F, F3), lambda i: (0, 0)),
            pl.BlockSpec((1, F3), lambda i: (0, 0)),
        ],
        out_specs=pl.BlockSpec((_PHI_BLK, F3), lambda i: (i, 0)),
        out_shape=jax.ShapeDtypeStruct((N, F3), jnp.float32),
    )(s, W1, b1.reshape(1, F), W2, b2.reshape(1, F3))


_NC = 2
_NS = 16
_NW = _NC * _NS
_EG = E // _NW
_CH_G = 40


def _gather_body(phi_hbm, vec_hbm, dst_hbm, gphi_hbm, gvec_hbm,
                 idx_v, bphi, bvec, sem):
    wid = lax.axis_index("s") * _NC + lax.axis_index("c")
    base = wid * _EG

    @pl.loop(0, _EG // _CH_G)
    def _(i):
        off = base + i * _CH_G
        pltpu.sync_copy(dst_hbm.at[pl.ds(off, _CH_G)], idx_v)
        pltpu.async_copy(phi_hbm.at[idx_v], bphi, sem).wait()
        pltpu.sync_copy(bphi, gphi_hbm.at[pl.ds(off, _CH_G)])
        pltpu.async_copy(vec_hbm.at[idx_v], bvec, sem).wait()
        pltpu.sync_copy(bvec, gvec_hbm.at[pl.ds(off, _CH_G)])


def _gather_sc(phi, vec2, dst):
    k = pl.kernel(
        _gather_body,
        out_type=[
            jax.ShapeDtypeStruct((E, F3), jnp.float32),
            jax.ShapeDtypeStruct((E, F3), jnp.float32),
        ],
        mesh=plsc.VectorSubcoreMesh(core_axis_name="c", subcore_axis_name="s"),
        scratch_types=[
            pltpu.VMEM((_CH_G,), jnp.int32),
            pltpu.VMEM((_CH_G, F3), jnp.float32),
            pltpu.VMEM((_CH_G, F3), jnp.float32),
            pltpu.SemaphoreType.DMA,
        ],
    )
    return k(phi, vec2, dst)


_EDGE_BLK = 2000


def _edge_body(cut_ref, gphi_ref, gvec_ref, rbf_ref, geom_ref, wr_ref, br_ref,
               o_ref):
    cut = cut_ref[0, 0]
    d = geom_ref[:, 0:1]
    fcut = 0.5 * (jnp.cos(np.pi * d / cut) + 1.0)
    fcut = jnp.where(d < cut, fcut, 0.0)
    w = jnp.dot(rbf_ref[...], wr_ref[...], preferred_element_type=jnp.float32)
    w = (w + br_ref[...]) * fcut
    x = gphi_ref[...] * w
    x_lo = x[:, 0:F]
    x_mid = x[:, F:2 * F]
    x_hi = x[:, 2 * F:3 * F]
    inv_d = 1.0 / d
    parts = [x_lo]
    for k in range(3):
        nk = geom_ref[:, k + 1:k + 2] * inv_d
        parts.append(x_mid * gvec_ref[:, F * k:F * (k + 1)] + nk * x_hi)
    o_ref[...] = jnp.concatenate(parts, axis=1)


def _edge_tc(cut_arr, gphi, gvec, edge_rbf, geom, Wr, br):
    n_rbf = edge_rbf.shape[1]
    return pl.pallas_call(
        _edge_body,
        grid=(E // _EDGE_BLK,),
        in_specs=[
            pl.BlockSpec((1, 1), lambda i: (0, 0)),
            pl.BlockSpec((_EDGE_BLK, F3), lambda i: (i, 0)),
            pl.BlockSpec((_EDGE_BLK, F3), lambda i: (i, 0)),
            pl.BlockSpec((_EDGE_BLK, n_rbf), lambda i: (i, 0)),
            pl.BlockSpec((_EDGE_BLK, 4), lambda i: (i, 0)),
            pl.BlockSpec((n_rbf, F3), lambda i: (0, 0)),
            pl.BlockSpec((1, F3), lambda i: (0, 0)),
        ],
        out_specs=pl.BlockSpec((_EDGE_BLK, 4 * F), lambda i: (i, 0)),
        out_shape=jax.ShapeDtypeStruct((E, 4 * F), jnp.float32),
    )(cut_arr, gphi, gvec, edge_rbf, geom, Wr, br.reshape(1, F3))


_ES = E // _NS
_CH_S = 80


def _scatter_body(pay_hbm, src_hbm, zero_hbm, ds_hbm, dvec_hbm,
                  idx_v, buf, accum, sem):
    c = lax.axis_index("c")
    sid = lax.axis_index("s")
    base = sid * _ES

    for blk in range(4):
        @pl.when(c == (blk % 2))
        def _():
            plsc.subcore_barrier()

            @pl.when(sid == 0)
            def _():
                pltpu.sync_copy(zero_hbm, accum)

            plsc.subcore_barrier()

            @pl.loop(0, _ES // _CH_S)
            def _(i):
                off = base + i * _CH_S
                pltpu.sync_copy(src_hbm.at[pl.ds(off, _CH_S)], idx_v)
                pltpu.sync_copy(
                    pay_hbm.at[pl.ds(off, _CH_S), pl.ds(blk * F, F)], buf)
                pltpu.sync_copy(buf, accum.at[idx_v], add=True)

            plsc.subcore_barrier()

            @pl.when(sid == 0)
            def _():
                if blk == 0:
                    pltpu.sync_copy(accum, ds_hbm)
                else:
                    pltpu.sync_copy(accum, dvec_hbm.at[blk - 1])


def _scatter_sc(pay, src, zeros):
    k = pl.kernel(
        _scatter_body,
        out_type=[
            jax.ShapeDtypeStruct((N, F), jnp.float32),
            jax.ShapeDtypeStruct((3, N, F), jnp.float32),
        ],
        mesh=plsc.VectorSubcoreMesh(core_axis_name="c", subcore_axis_name="s"),
        scratch_types=[
            pltpu.VMEM((_CH_S,), jnp.int32),
            pltpu.VMEM((_CH_S, F), jnp.float32),
            pltpu.VMEM_SHARED((N, F), jnp.float32),
            pltpu.SemaphoreType.DMA,
        ],
    )
    return k(pay, src, zeros)


def kernel(s, vec, edge_indexes, edge_vector, edge_distance, edge_rbf,
           cutoff_dist, W1, b1, W2, b2, Wr, br):
    src = edge_indexes[0].astype(jnp.int32)
    dst = edge_indexes[1].astype(jnp.int32)
    vec2 = vec.reshape(N, F3)
    cut_arr = jnp.asarray(cutoff_dist, jnp.float32).reshape(1, 1)
    geom = jnp.concatenate(
        [edge_distance.reshape(E, 1), edge_vector], axis=1)
    zeros = jnp.zeros((N, F), jnp.float32)

    phi = _phi_tc(s, W1, b1, W2, b2)
    gphi, gvec = _gather_sc(phi, vec2, dst)
    pay = _edge_tc(cut_arr, gphi, gvec, edge_rbf, geom, Wr, br)
    ds, dvec = _scatter_sc(pay, src, zeros)
    return ds, dvec.transpose(1, 0, 2)


# speedup vs baseline: 15.8175x; 1.3262x over previous
import functools

import jax
import jax.numpy as jnp
import numpy as np
from jax import lax
from jax.experimental import pallas as pl
from jax.experimental.pallas import tpu as pltpu
from jax.experimental.pallas import tpu_sc as plsc

N = 10000
E = 160000
F = 128
F3 = 3 * F

_NC = 2
_NS = 16
_ES = E // _NS
_CH = 80
_NCH = _ES // _CH
_NPAIR = _NCH // 2


_PHI_BLK = 1000


def _phi_body(s_ref, w1_ref, b1_ref, w2_ref, b2_ref, o_ref):
    h = jnp.dot(s_ref[...], w1_ref[...], preferred_element_type=jnp.float32)
    h = jax.nn.silu(h + b1_ref[...])
    o_ref[...] = (
        jnp.dot(h, w2_ref[...], preferred_element_type=jnp.float32) + b2_ref[...]
    )


def _phi_tc(s, W1, b1, W2, b2):
    return pl.pallas_call(
        _phi_body,
        grid=(N // _PHI_BLK,),
        in_specs=[
            pl.BlockSpec((_PHI_BLK, F), lambda i: (i, 0)),
            pl.BlockSpec((F, F), lambda i: (0, 0)),
            pl.BlockSpec((1, F), lambda i: (0, 0)),
            pl.BlockSpec((F, F3), lambda i: (0, 0)),
            pl.BlockSpec((1, F3), lambda i: (0, 0)),
        ],
        out_specs=pl.BlockSpec((_PHI_BLK, F3), lambda i: (i, 0)),
        out_shape=jax.ShapeDtypeStruct((N, F3), jnp.float32),
    )(s, W1, b1.reshape(1, F), W2, b2.reshape(1, F3))


def _gather_stream(tbl_hbm, out_hbm, idx_v, bufs, gsems, csems, base):
    def chunk_start(i, b):
        return pltpu.async_copy(
            tbl_hbm.at[idx_v.at[pl.ds(i * _CH, _CH)]], bufs[b], gsems[b])

    def chunk_out(i, b):
        pltpu.async_copy(bufs[b], out_hbm.at[pl.ds(base + i * _CH, _CH)],
                         csems[b])

    def drain_out(i, b):
        pltpu.make_async_copy(
            bufs[b], out_hbm.at[pl.ds(base + i * _CH, _CH)], csems[b]).wait()

    @pl.loop(0, _NPAIR)
    def _(j):
        i0 = 2 * j
        for b in (0, 1):
            @pl.when(j > 0)
            def _():
                drain_out(i0 + b - 2, b)
            chunk_start(i0 + b, b)
        for b in (0, 1):
            pltpu.make_async_copy(
                tbl_hbm.at[idx_v.at[pl.ds((i0 + b) * _CH, _CH)]],
                bufs[b], gsems[b]).wait()
            chunk_out(i0 + b, b)

    tail = _NCH - 1
    drain_out(tail - 2, 0)
    chunk_start(tail, 0).wait()
    chunk_out(tail, 0)
    drain_out(tail, 0)
    drain_out(tail - 1, 1)


def _gather_body(phi_hbm, vec_hbm, dst_hbm, gphi_hbm, gvec_hbm,
                 idx_v, b0, b1, gsem0, gsem1, csem0, csem1):
    c = lax.axis_index("c")
    sid = lax.axis_index("s")
    base = sid * _ES
    pltpu.sync_copy(dst_hbm.at[pl.ds(base, _ES)], idx_v)

    @pl.when(c == 0)
    def _():
        _gather_stream(phi_hbm, gphi_hbm, idx_v, (b0, b1),
                       (gsem0, gsem1), (csem0, csem1), base)

    @pl.when(c == 1)
    def _():
        _gather_stream(vec_hbm, gvec_hbm, idx_v, (b0, b1),
                       (gsem0, gsem1), (csem0, csem1), base)


def _gather_sc(phi, vec2, dst):
    k = pl.kernel(
        _gather_body,
        out_type=[
            jax.ShapeDtypeStruct((E, F3), jnp.float32),
            jax.ShapeDtypeStruct((E, F3), jnp.float32),
        ],
        mesh=plsc.VectorSubcoreMesh(core_axis_name="c", subcore_axis_name="s"),
        scratch_types=[
            pltpu.VMEM((_ES,), jnp.int32),
            pltpu.VMEM((_CH, F3), jnp.float32),
            pltpu.VMEM((_CH, F3), jnp.float32),
            pltpu.SemaphoreType.DMA,
            pltpu.SemaphoreType.DMA,
            pltpu.SemaphoreType.DMA,
            pltpu.SemaphoreType.DMA,
        ],
    )
    return k(phi, vec2, dst)


_EDGE_BLK = 2000


def _edge_body(cut_ref, gphi_ref, gvec_ref, rbf_ref, geom_ref, wr_ref, br_ref,
               o_ref):
    cut = cut_ref[0, 0]
    d = geom_ref[:, 0:1]
    fcut = 0.5 * (jnp.cos(np.pi * d / cut) + 1.0)
    fcut = jnp.where(d < cut, fcut, 0.0)
    w = jnp.dot(rbf_ref[...], wr_ref[...], preferred_element_type=jnp.float32)
    w = (w + br_ref[...]) * fcut
    x = gphi_ref[...] * w
    x_lo = x[:, 0:F]
    x_mid = x[:, F:2 * F]
    x_hi = x[:, 2 * F:3 * F]
    inv_d = 1.0 / d
    parts = [x_lo]
    for k in range(3):
        nk = geom_ref[:, k + 1:k + 2] * inv_d
        parts.append(x_mid * gvec_ref[:, F * k:F * (k + 1)] + nk * x_hi)
    o_ref[...] = jnp.concatenate(parts, axis=1)


def _edge_tc(cut_arr, gphi, gvec, edge_rbf, geom, Wr, br):
    n_rbf = edge_rbf.shape[1]
    return pl.pallas_call(
        _edge_body,
        grid=(E // _EDGE_BLK,),
        in_specs=[
            pl.BlockSpec((1, 1), lambda i: (0, 0)),
            pl.BlockSpec((_EDGE_BLK, F3), lambda i: (i, 0)),
            pl.BlockSpec((_EDGE_BLK, F3), lambda i: (i, 0)),
            pl.BlockSpec((_EDGE_BLK, n_rbf), lambda i: (i, 0)),
            pl.BlockSpec((_EDGE_BLK, 4), lambda i: (i, 0)),
            pl.BlockSpec((n_rbf, F3), lambda i: (0, 0)),
            pl.BlockSpec((1, F3), lambda i: (0, 0)),
        ],
        out_specs=pl.BlockSpec((_EDGE_BLK, 4 * F), lambda i: (i, 0)),
        out_shape=jax.ShapeDtypeStruct((E, 4 * F), jnp.float32),
    )(cut_arr, gphi, gvec, edge_rbf, geom, Wr, br.reshape(1, F3))


def _scatter_stream(pay_hbm, accum, idx2, bufs, lsems, ssems, base, blk):
    col = pl.ds(blk * F, F)

    def load_start(i, b):
        return pltpu.async_copy(
            pay_hbm.at[pl.ds(base + i * _CH, _CH), col], bufs[b], lsems[b])

    def scat_start(i, b):
        pltpu.async_copy(bufs[b], accum.at[idx2.at[i]], ssems[b], add=True)

    def drain_scat(i, b):
        pltpu.make_async_copy(
            bufs[b], accum.at[idx2.at[i]], ssems[b]).wait()

    @pl.loop(0, _NPAIR)
    def _(j):
        i0 = 2 * j
        for b in (0, 1):
            @pl.when(j > 0)
            def _():
                drain_scat(i0 + b - 2, b)
            load_start(i0 + b, b)
        for b in (0, 1):
            pltpu.make_async_copy(
                pay_hbm.at[pl.ds(base + (i0 + b) * _CH, _CH), col],
                bufs[b], lsems[b]).wait()
            scat_start(i0 + b, b)

    tail = _NCH - 1
    drain_scat(tail - 2, 0)
    load_start(tail, 0).wait()
    scat_start(tail, 0)
    drain_scat(tail, 0)
    drain_scat(tail - 1, 1)


def _scatter_body(pay_hbm, src2_hbm, zero_hbm, ds_hbm, dvec_hbm,
                  idx2, b0, b1, accum, lsem0, lsem1, ssem0, ssem1):
    c = lax.axis_index("c")
    sid = lax.axis_index("s")
    base = sid * _ES
    pltpu.sync_copy(src2_hbm.at[sid], idx2)

    for blk in range(4):
        @pl.when(c == (blk % 2))
        def _():
            plsc.subcore_barrier()

            @pl.when(sid == 0)
            def _():
                pltpu.sync_copy(zero_hbm, accum)

            plsc.subcore_barrier()
            _scatter_stream(pay_hbm, accum, idx2, (b0, b1),
                            (lsem0, lsem1), (ssem0, ssem1), base, blk)
            plsc.subcore_barrier()

            @pl.when(sid == 0)
            def _():
                if blk == 0:
                    pltpu.sync_copy(accum, ds_hbm)
                else:
                    pltpu.sync_copy(accum, dvec_hbm.at[blk - 1])


def _scatter_sc(pay, src2, zeros):
    k = pl.kernel(
        _scatter_body,
        out_type=[
            jax.ShapeDtypeStruct((N, F), jnp.float32),
            jax.ShapeDtypeStruct((3, N, F), jnp.float32),
        ],
        mesh=plsc.VectorSubcoreMesh(core_axis_name="c", subcore_axis_name="s"),
        scratch_types=[
            pltpu.VMEM((_NCH, _CH), jnp.int32),
            pltpu.VMEM((_CH, F), jnp.float32),
            pltpu.VMEM((_CH, F), jnp.float32),
            pltpu.VMEM_SHARED((N, F), jnp.float32),
            pltpu.SemaphoreType.DMA,
            pltpu.SemaphoreType.DMA,
            pltpu.SemaphoreType.DMA,
            pltpu.SemaphoreType.DMA,
        ],
    )
    return k(pay, src2, zeros)


def kernel(s, vec, edge_indexes, edge_vector, edge_distance, edge_rbf,
           cutoff_dist, W1, b1, W2, b2, Wr, br):
    src2 = edge_indexes[0].astype(jnp.int32).reshape(_NS, _NCH, _CH)
    dst = edge_indexes[1].astype(jnp.int32)
    vec2 = vec.reshape(N, F3)
    cut_arr = jnp.asarray(cutoff_dist, jnp.float32).reshape(1, 1)
    geom = jnp.concatenate(
        [edge_distance.reshape(E, 1), edge_vector], axis=1)
    zeros = jnp.zeros((N, F), jnp.float32)

    phi = _phi_tc(s, W1, b1, W2, b2)
    gphi, gvec = _gather_sc(phi, vec2, dst)
    pay = _edge_tc(cut_arr, gphi, gvec, edge_rbf, geom, Wr, br)
    ds, dvec = _scatter_sc(pay, src2, zeros)
    return ds, dvec.transpose(1, 0, 2)


# speedup vs baseline: 16.4705x; 1.0413x over previous
import functools

import jax
import jax.numpy as jnp
import numpy as np
from jax import lax
from jax.experimental import pallas as pl
from jax.experimental.pallas import tpu as pltpu
from jax.experimental.pallas import tpu_sc as plsc

N = 10000
E = 160000
F = 128
F3 = 3 * F

_NC = 2
_NS = 16
_NHALF = 2
_EH = E // _NHALF
_ET = _EH // _NS
_CHG = 40
_NCHG = _ET // _CHG

_ES = E // _NS
_CH = 80
_NCH = _ES // _CH

_TC_PARAMS = pltpu.CompilerParams(dimension_semantics=("parallel",))


_PHI_BLK = 1000


def _phi_body(s_ref, w1_ref, b1_ref, w2_ref, b2_ref, o_ref):
    h = jnp.dot(s_ref[...], w1_ref[...], preferred_element_type=jnp.float32)
    h = jax.nn.silu(h + b1_ref[...])
    o_ref[...] = (
        jnp.dot(h, w2_ref[...], preferred_element_type=jnp.float32) + b2_ref[...]
    )


def _phi_tc(s, W1, b1, W2, b2):
    return pl.pallas_call(
        _phi_body,
        grid=(N // _PHI_BLK,),
        in_specs=[
            pl.BlockSpec((_PHI_BLK, F), lambda i: (i, 0)),
            pl.BlockSpec((F, F), lambda i: (0, 0)),
            pl.BlockSpec((1, F), lambda i: (0, 0)),
            pl.BlockSpec((F, F3), lambda i: (0, 0)),
            pl.BlockSpec((1, F3), lambda i: (0, 0)),
        ],
        out_specs=pl.BlockSpec((_PHI_BLK, F3), lambda i: (i, 0)),
        out_shape=jax.ShapeDtypeStruct((N, F3), jnp.float32),
        compiler_params=_TC_PARAMS,
    )(s, W1, b1.reshape(1, F), W2, b2.reshape(1, F3))


def _gather_stream(tbl_hbm, out_hbm, idx_v, bufs, gsems, csems, base):
    def chunk_start(i, b):
        return pltpu.async_copy(
            tbl_hbm.at[idx_v.at[pl.ds(i * _CHG, _CHG)]], bufs[b], gsems[b])

    def chunk_out(i, b):
        pltpu.async_copy(bufs[b], out_hbm.at[pl.ds(base + i * _CHG, _CHG)],
                         csems[b])

    def drain_out(i, b):
        pltpu.make_async_copy(
            bufs[b], out_hbm.at[pl.ds(base + i * _CHG, _CHG)], csems[b]).wait()

    @pl.loop(0, _NCHG // 2)
    def _(j):
        i0 = 2 * j
        for b in (0, 1):
            @pl.when(j > 0)
            def _():
                drain_out(i0 + b - 2, b)
            chunk_start(i0 + b, b)
        for b in (0, 1):
            pltpu.make_async_copy(
                tbl_hbm.at[idx_v.at[pl.ds((i0 + b) * _CHG, _CHG)]],
                bufs[b], gsems[b]).wait()
            chunk_out(i0 + b, b)

    tail = _NCHG - 1
    drain_out(tail - 2, 0)
    chunk_start(tail, 0).wait()
    chunk_out(tail, 0)
    drain_out(tail, 0)
    drain_out(tail - 1, 1)


def _gather_body(phi_hbm, vec_hbm, dst_hbm, gphi_hbm, gvec_hbm,
                 idx_v, b0, b1, gsem0, gsem1, csem0, csem1):
    c = lax.axis_index("c")
    sid = lax.axis_index("s")
    base = sid * _ET
    pltpu.sync_copy(dst_hbm.at[pl.ds(base, _ET)], idx_v)

    @pl.when(c == 0)
    def _():
        _gather_stream(phi_hbm, gphi_hbm, idx_v, (b0, b1),
                       (gsem0, gsem1), (csem0, csem1), base)

    @pl.when(c == 1)
    def _():
        _gather_stream(vec_hbm, gvec_hbm, idx_v, (b0, b1),
                       (gsem0, gsem1), (csem0, csem1), base)


def _gather_sc(phi, vec2, dst_half):
    k = pl.kernel(
        _gather_body,
        out_type=[
            jax.ShapeDtypeStruct((_EH, F3), jnp.float32),
            jax.ShapeDtypeStruct((_EH, F3), jnp.float32),
        ],
        mesh=plsc.VectorSubcoreMesh(core_axis_name="c", subcore_axis_name="s"),
        scratch_types=[
            pltpu.VMEM((_ET,), jnp.int32),
            pltpu.VMEM((_CHG, F3), jnp.float32),
            pltpu.VMEM((_CHG, F3), jnp.float32),
            pltpu.SemaphoreType.DMA,
            pltpu.SemaphoreType.DMA,
            pltpu.SemaphoreType.DMA,
            pltpu.SemaphoreType.DMA,
        ],
    )
    return k(phi, vec2, dst_half)


_EDGE_BLK = 2000


def _edge_body(cut_ref, gphi_ref, gvec_ref, rbf_ref, geom_ref, wr_ref, br_ref,
               o_ref):
    cut = cut_ref[0, 0]
    d = geom_ref[:, 0:1]
    fcut = 0.5 * (jnp.cos(np.pi * d / cut) + 1.0)
    fcut = jnp.where(d < cut, fcut, 0.0)
    w = jnp.dot(rbf_ref[...], wr_ref[...], preferred_element_type=jnp.float32)
    w = (w + br_ref[...]) * fcut
    x = gphi_ref[...] * w
    x_lo = x[:, 0:F]
    x_mid = x[:, F:2 * F]
    x_hi = x[:, 2 * F:3 * F]
    inv_d = 1.0 / d
    parts = [x_lo]
    for k in range(3):
        nk = geom_ref[:, k + 1:k + 2] * inv_d
        parts.append(x_mid * gvec_ref[:, F * k:F * (k + 1)] + nk * x_hi)
    o_ref[...] = jnp.concatenate(parts, axis=1)


def _edge_tc(cut_arr, gphi, gvec, edge_rbf, geom, Wr, br2):
    n_rbf = edge_rbf.shape[1]
    return pl.pallas_call(
        _edge_body,
        grid=(_EH // _EDGE_BLK,),
        in_specs=[
            pl.BlockSpec((1, 1), lambda i: (0, 0)),
            pl.BlockSpec((_EDGE_BLK, F3), lambda i: (i, 0)),
            pl.BlockSpec((_EDGE_BLK, F3), lambda i: (i, 0)),
            pl.BlockSpec((_EDGE_BLK, n_rbf), lambda i: (i, 0)),
            pl.BlockSpec((_EDGE_BLK, 4), lambda i: (i, 0)),
            pl.BlockSpec((n_rbf, F3), lambda i: (0, 0)),
            pl.BlockSpec((1, F3), lambda i: (0, 0)),
        ],
        out_specs=pl.BlockSpec((_EDGE_BLK, 4 * F), lambda i: (i, 0)),
        out_shape=jax.ShapeDtypeStruct((_EH, 4 * F), jnp.float32),
        compiler_params=_TC_PARAMS,
    )(cut_arr, gphi, gvec, edge_rbf, geom, Wr, br2)


def _scatter_stream(pay_hbm, accum, idx2, bufs, lsems, ssems, base, blk):
    col = pl.ds(blk * F, F)

    def load_start(i, b):
        return pltpu.async_copy(
            pay_hbm.at[pl.ds(base + i * _CH, _CH), col], bufs[b], lsems[b])

    def scat_start(i, b):
        pltpu.async_copy(bufs[b], accum.at[idx2.at[i]], ssems[b], add=True)

    def drain_scat(i, b):
        pltpu.make_async_copy(
            bufs[b], accum.at[idx2.at[i]], ssems[b]).wait()

    @pl.loop(0, _NCH // 2)
    def _(j):
        i0 = 2 * j
        for b in (0, 1):
            @pl.when(j > 0)
            def _():
                drain_scat(i0 + b - 2, b)
            load_start(i0 + b, b)
        for b in (0, 1):
            pltpu.make_async_copy(
                pay_hbm.at[pl.ds(base + (i0 + b) * _CH, _CH), col],
                bufs[b], lsems[b]).wait()
            scat_start(i0 + b, b)

    tail = _NCH - 1
    drain_scat(tail - 2, 0)
    load_start(tail, 0).wait()
    scat_start(tail, 0)
    drain_scat(tail, 0)
    drain_scat(tail - 1, 1)


def _scatter_body(pay0_hbm, pay1_hbm, src3_hbm, zero_hbm, ds_hbm, dvec_hbm,
                  idx2, b0, b1, accum, lsem0, lsem1, ssem0, ssem1):
    c = lax.axis_index("c")
    sid = lax.axis_index("s")
    pltpu.sync_copy(src3_hbm.at[sid], idx2)
    half = _NS // 2

    for blk in range(4):
        @pl.when(c == (blk % 2))
        def _():
            plsc.subcore_barrier()

            @pl.when(sid == 0)
            def _():
                pltpu.sync_copy(zero_hbm, accum)

            plsc.subcore_barrier()

            @pl.when(sid < half)
            def _():
                _scatter_stream(pay0_hbm, accum, idx2, (b0, b1),
                                (lsem0, lsem1), (ssem0, ssem1),
                                sid * _ES, blk)

            @pl.when(sid >= half)
            def _():
                _scatter_stream(pay1_hbm, accum, idx2, (b0, b1),
                                (lsem0, lsem1), (ssem0, ssem1),
                                (sid - half) * _ES, blk)

            plsc.subcore_barrier()

            @pl.when(sid == 0)
            def _():
                if blk == 0:
                    pltpu.sync_copy(accum, ds_hbm)
                else:
                    pltpu.sync_copy(accum, dvec_hbm.at[blk - 1])


def _scatter_sc(pay0, pay1, src3, zeros):
    k = pl.kernel(
        _scatter_body,
        out_type=[
            jax.ShapeDtypeStruct((N, F), jnp.float32),
            jax.ShapeDtypeStruct((3, N, F), jnp.float32),
        ],
        mesh=plsc.VectorSubcoreMesh(core_axis_name="c", subcore_axis_name="s"),
        scratch_types=[
            pltpu.VMEM((_NCH, _CH), jnp.int32),
            pltpu.VMEM((_CH, F), jnp.float32),
            pltpu.VMEM((_CH, F), jnp.float32),
            pltpu.VMEM_SHARED((N, F), jnp.float32),
            pltpu.SemaphoreType.DMA,
            pltpu.SemaphoreType.DMA,
            pltpu.SemaphoreType.DMA,
            pltpu.SemaphoreType.DMA,
        ],
    )
    return k(pay0, pay1, src3, zeros)


def kernel(s, vec, edge_indexes, edge_vector, edge_distance, edge_rbf,
           cutoff_dist, W1, b1, W2, b2, Wr, br):
    src3 = edge_indexes[0].astype(jnp.int32).reshape(_NS, _NCH, _CH)
    dst = edge_indexes[1].astype(jnp.int32)
    vec2 = vec.reshape(N, F3)
    cut_arr = jnp.asarray(cutoff_dist, jnp.float32).reshape(1, 1)
    geom = jnp.concatenate(
        [edge_distance.reshape(E, 1), edge_vector], axis=1)
    zeros = jnp.zeros((N, F), jnp.float32)
    br2 = br.reshape(1, F3)

    phi = _phi_tc(s, W1, b1, W2, b2)
    pay_halves = []
    for h in range(_NHALF):
        rows = slice(h * _EH, (h + 1) * _EH)
        gphi, gvec = _gather_sc(phi, vec2, dst[rows])
        pay_halves.append(
            _edge_tc(cut_arr, gphi, gvec, edge_rbf[rows], geom[rows], Wr, br2))
    ds, dvec = _scatter_sc(pay_halves[0], pay_halves[1], src3, zeros)
    return ds, dvec.transpose(1, 0, 2)
